# Initial kernel scaffold; baseline (speedup 1.0000x reference)
import jax
import jax.numpy as jnp
from jax.experimental import pallas as pl


def kernel(flat_tokens, cu_seqlens, max_seq_len):
    raise NotImplementedError("write your pallas kernel here")


# speedup vs baseline: 13.4209x; 13.4209x over previous
import functools

import jax
import jax.numpy as jnp
from jax import lax
from jax.experimental import pallas as pl
from jax.experimental.pallas import tpu as pltpu
from jax.experimental.pallas import tpu_sc as plsc

START_TOKEN = 101
END_TOKEN = 102
TOTAL_TOK = 32768
BATCH = 16
L_OUT = 4098
HALF = 2064
W_PAD = 2 * HALF
C_IN = 2072
GUARD = 8
TOK_BUF = 4160
BASE_MAX = GUARD + (TOTAL_TOK - 1) - (TOTAL_TOK - C_IN) + 1
N_GROUPS = HALF // 16

_mesh = plsc.VectorSubcoreMesh(core_axis_name="c", subcore_axis_name="s")


@functools.partial(
    pl.kernel,
    out_type=jax.ShapeDtypeStruct((BATCH * W_PAD,), jnp.int32),
    mesh=_mesh,
    scratch_types=[
        pltpu.VMEM((64,), jnp.int32),
        pltpu.VMEM((TOK_BUF,), jnp.int32),
        pltpu.VMEM((HALF,), jnp.int32),
    ],
)
def _sc_body(flat_hbm, meta_hbm, out_hbm, meta_v, tok_v, row_v):
    row = lax.axis_index("s")
    half = lax.axis_index("c")
    lanes = lax.iota(jnp.int32, 16)

    pltpu.sync_copy(meta_hbm, meta_v.at[pl.ds(0, 48)])
    start = meta_v[pl.ds(row, 16)][0]
    end = meta_v[pl.ds(row + 16, 16)][0]
    msl = meta_v[pl.ds(row + 32, 16)][0]
    trim = jnp.minimum(end - start, msl)

    p0 = half * HALF
    t0 = jnp.maximum(start + p0 - 1, 0)
    s_al = jnp.minimum((t0 // 8) * 8, TOTAL_TOK - C_IN)
    s_al = pl.multiple_of(s_al, 8)
    pltpu.sync_copy(flat_hbm.at[pl.ds(s_al, C_IN)], tok_v.at[pl.ds(GUARD, C_IN)])
    base0 = jnp.minimum(GUARD + start + p0 - 1 - s_al, BASE_MAX)

    def step(i, carry):
        p = p0 + i * 16 + lanes
        vals = tok_v[pl.ds(base0 + i * 16, 16)]
        o = jnp.where(p == 0, START_TOKEN,
             jnp.where(p == trim + 1, END_TOKEN,
              jnp.where(p <= trim, vals, 0)))
        row_v[pl.ds(i * 16, 16)] = o
        return carry

    lax.fori_loop(0, N_GROUPS, step, 0)
    dst = pl.multiple_of(row * W_PAD + p0, 8)
    pltpu.sync_copy(row_v, out_hbm.at[pl.ds(dst, HALF)])


def kernel(flat_tokens, cu_seqlens, max_seq_len):
    starts = cu_seqlens[:-1].astype(jnp.int32)
    ends = cu_seqlens[1:].astype(jnp.int32)
    msl = jnp.broadcast_to(jnp.asarray(max_seq_len, jnp.int32), (BATCH,))
    meta = jnp.concatenate([starts, ends, msl])
    out_flat = _sc_body(flat_tokens.astype(jnp.int32), meta)
    input_ids = out_flat.reshape(BATCH, W_PAD)[:, :L_OUT]
    token_type_ids = jnp.zeros((BATCH, L_OUT), jnp.int32)
    return (input_ids, token_type_ids)


# speedup vs baseline: 13.5069x; 1.0064x over previous
import functools

import jax
import jax.numpy as jnp
from jax import lax
from jax.experimental import pallas as pl
from jax.experimental.pallas import tpu as pltpu
from jax.experimental.pallas import tpu_sc as plsc

START_TOKEN = 101
END_TOKEN = 102
TOTAL_TOK = 32768
BATCH = 16
L_OUT = 4098
HALF = 2064
TAIL = L_OUT - HALF
C_IN = 2072
GUARD = 8
TOK_BUF = 4160
BASE_MAX = GUARD + (TOTAL_TOK - 1) - (TOTAL_TOK - C_IN) + 1

_mesh = plsc.VectorSubcoreMesh(core_axis_name="c", subcore_axis_name="s")


@functools.partial(
    pl.kernel,
    out_type=jax.ShapeDtypeStruct((BATCH, L_OUT), jnp.int32),
    mesh=_mesh,
    compiler_params=pltpu.CompilerParams(use_tc_tiling_on_sc=False),
    scratch_types=[
        pltpu.VMEM((32,), jnp.int32),
        pltpu.VMEM((16,), jnp.int32),
        pltpu.VMEM((TOK_BUF,), jnp.int32),
        pltpu.VMEM((HALF,), jnp.int32),
    ],
)
def _sc_body(flat_hbm, cu_hbm, msl_hbm, out_hbm, cu_v, msl_v, tok_v, row_v):
    row = lax.axis_index("s")
    half = lax.axis_index("c")
    lanes = lax.iota(jnp.int32, 16)

    pltpu.sync_copy(cu_hbm.at[pl.ds(0, 16)], cu_v.at[pl.ds(0, 16)])
    pltpu.sync_copy(msl_hbm, msl_v.at[pl.ds(0, 8)])
    start = cu_v[pl.ds(row, 16)][0]
    end = jnp.where(row == BATCH - 1, TOTAL_TOK, cu_v[pl.ds(row + 1, 16)][0])
    msl = msl_v[pl.ds(0, 16)][0]
    trim = jnp.minimum(end - start, msl)

    p0 = half * HALF
    t0 = jnp.maximum(start + p0 - 1, 0)
    s_al = jnp.minimum((t0 // 8) * 8, TOTAL_TOK - C_IN)
    s_al = pl.multiple_of(s_al, 8)
    pltpu.sync_copy(flat_hbm.at[pl.ds(s_al, C_IN)], tok_v.at[pl.ds(GUARD, C_IN)])
    base0 = jnp.minimum(GUARD + start + p0 - 1 - s_al, BASE_MAX)

    n_groups = jnp.where(half == 0, HALF // 16, (L_OUT - HALF + 15) // 16)

    def step(i, carry):
        p = p0 + i * 16 + lanes
        vals = tok_v[pl.ds(base0 + i * 16, 16)]
        o = jnp.where(p == 0, START_TOKEN,
             jnp.where(p == trim + 1, END_TOKEN,
              jnp.where(p <= trim, vals, 0)))
        row_v[pl.ds(i * 16, 16)] = o
        return carry

    lax.fori_loop(0, n_groups, step, 0)

    @pl.when(half == 0)
    def _():
        pltpu.sync_copy(row_v, out_hbm.at[row, pl.ds(0, HALF)])

    @pl.when(half == 1)
    def _():
        pltpu.sync_copy(row_v.at[pl.ds(0, TAIL)],
                        out_hbm.at[row, pl.ds(HALF, TAIL)])


def kernel(flat_tokens, cu_seqlens, max_seq_len):
    msl = jnp.broadcast_to(jnp.asarray(max_seq_len, jnp.int32), (8,))
    input_ids = _sc_body(flat_tokens.astype(jnp.int32),
                         cu_seqlens.astype(jnp.int32), msl)
    token_type_ids = jnp.zeros((BATCH, L_OUT), jnp.int32)
    return (input_ids, token_type_ids)


# speedup vs baseline: 13.5083x; 1.0001x over previous
import functools

import jax
import jax.numpy as jnp
from jax import lax
from jax.experimental import pallas as pl
from jax.experimental.pallas import tpu as pltpu
from jax.experimental.pallas import tpu_sc as plsc

START_TOKEN = 101
END_TOKEN = 102
TOTAL_TOK = 32768
BATCH = 16
L_OUT = 4098
HALF = 2064
TAIL = L_OUT - HALF
C_IN = 2072
GUARD = 8
TOK_BUF = 4160
BASE_MAX = GUARD + (TOTAL_TOK - 1) - (TOTAL_TOK - C_IN) + 1

_mesh = plsc.VectorSubcoreMesh(core_axis_name="c", subcore_axis_name="s")


@functools.partial(
    pl.kernel,
    out_type=jax.ShapeDtypeStruct((BATCH, L_OUT), jnp.int32),
    mesh=_mesh,
    compiler_params=pltpu.CompilerParams(use_tc_tiling_on_sc=False),
    scratch_types=[
        pltpu.VMEM((32,), jnp.int32),
        pltpu.VMEM((16,), jnp.int32),
        pltpu.VMEM((TOK_BUF,), jnp.int32),
        pltpu.VMEM((HALF,), jnp.int32),
    ],
)
def _sc_body(flat_hbm, cu_hbm, msl_hbm, out_hbm, cu_v, msl_v, tok_v, row_v):
    row = lax.axis_index("s")
    half = lax.axis_index("c")
    lanes = lax.iota(jnp.int32, 16)

    pltpu.sync_copy(cu_hbm.at[pl.ds(0, 16)], cu_v.at[pl.ds(0, 16)])
    pltpu.sync_copy(msl_hbm, msl_v.at[pl.ds(0, 8)])
    start = cu_v[pl.ds(row, 16)][0]
    end = jnp.where(row == BATCH - 1, TOTAL_TOK, cu_v[pl.ds(row + 1, 16)][0])
    msl = msl_v[pl.ds(0, 16)][0]
    trim = jnp.minimum(end - start, msl)

    p0 = half * HALF
    t0 = jnp.maximum(start + p0 - 1, 0)
    s_al = jnp.minimum((t0 // 8) * 8, TOTAL_TOK - C_IN)
    s_al = pl.multiple_of(s_al, 8)
    pltpu.sync_copy(flat_hbm.at[pl.ds(s_al, C_IN)], tok_v.at[pl.ds(GUARD, C_IN)])
    base0 = jnp.minimum(GUARD + start + p0 - 1 - s_al, BASE_MAX)

    n_groups = jnp.where(half == 0, HALF // 16, (L_OUT - HALF + 15) // 16)
    nc = jnp.clip((trim - p0 + 1) // 16, 0, n_groups)

    @plsc.parallel_loop(0, nc, unroll=4)
    def _copy(i):
        row_v[pl.ds(i * 16, 16)] = tok_v[pl.ds(base0 + i * 16, 16)]

    @pl.when(nc < n_groups)
    def _():
        p = p0 + nc * 16 + lanes
        vals = tok_v[pl.ds(base0 + nc * 16, 16)]
        o = jnp.where(p == 0, START_TOKEN,
             jnp.where(p == trim + 1, END_TOKEN,
              jnp.where(p <= trim, vals, 0)))
        row_v[pl.ds(nc * 16, 16)] = o

    zvec = jnp.zeros((16,), jnp.int32)

    @plsc.parallel_loop(jnp.minimum(nc + 1, n_groups), n_groups, unroll=4)
    def _zero(i):
        row_v[pl.ds(i * 16, 16)] = zvec

    @pl.when(half == 0)
    def _():
        row_v[pl.ds(0, 16)] = jnp.where(lanes == 0, START_TOKEN,
                                        row_v[pl.ds(0, 16)])

    @pl.when(half == 0)
    def _():
        pltpu.sync_copy(row_v, out_hbm.at[row, pl.ds(0, HALF)])

    @pl.when(half == 1)
    def _():
        pltpu.sync_copy(row_v.at[pl.ds(0, TAIL)],
                        out_hbm.at[row, pl.ds(HALF, TAIL)])


def kernel(flat_tokens, cu_seqlens, max_seq_len):
    msl = jnp.broadcast_to(jnp.asarray(max_seq_len, jnp.int32), (8,))
    input_ids = _sc_body(flat_tokens.astype(jnp.int32),
                         cu_seqlens.astype(jnp.int32), msl)
    token_type_ids = jnp.zeros((BATCH, L_OUT), jnp.int32)
    return (input_ids, token_type_ids)
